# Initial kernel scaffold; baseline (speedup 1.0000x reference)
import jax
import jax.numpy as jnp
from jax.experimental import pallas as pl


def kernel(x, edge_index, W, b):
    raise NotImplementedError("write your pallas kernel here")


# speedup vs baseline: 9.2204x; 9.2204x over previous
import functools

import jax
import jax.numpy as jnp
from jax import lax
from jax.experimental import pallas as pl
from jax.experimental.pallas import tpu as pltpu
from jax.experimental.pallas import tpu_sc as plsc

N = 10000
M = 128
NPAD = 10240
NCORES = 2
NSUB = 16
NW = NCORES * NSUB
RPT = NPAD // NW
CH = 128
EPT = 10240
NCH = EPT // CH
EPAD = NW * EPT
NCHD = 2 * NCH
GRP = 8

_mesh = plsc.VectorSubcoreMesh(
    core_axis_name="c", subcore_axis_name="s", num_cores=NCORES, num_subcores=NSUB
)


@functools.partial(
    pl.kernel,
    out_type=jax.ShapeDtypeStruct((NPAD,), jnp.float32),
    mesh=_mesh,
    scratch_types=[
        pltpu.VMEM_SHARED((NSUB * NPAD,), jnp.float32),
        pltpu.VMEM((NCHD, CH), jnp.int32),
        pltpu.VMEM((CH,), jnp.int32),
        pltpu.VMEM((CH,), jnp.float32),
        pltpu.VMEM((RPT,), jnp.float32),
        pltpu.VMEM((NSUB * RPT,), jnp.float32),
    ],
)
def _deg(dstd_hbm, deg_hbm, deg_sp, dstv, idxv, ones_v, dv, sumv):
    c = lax.axis_index("c")
    s = lax.axis_index("s")
    wid = c * NSUB + s
    for k in range(RPT // 16):
        dv[pl.ds(k * 16, 16)] = jnp.zeros((16,), jnp.float32)
    for j in range(NPAD // RPT):
        pltpu.sync_copy(dv, deg_sp.at[pl.ds(s * NPAD + j * RPT, RPT)])
    for k in range(CH // 16):
        ones_v[pl.ds(k * 16, 16)] = jnp.ones((16,), jnp.float32)
    pltpu.sync_copy(dstd_hbm.at[s], dstv)

    def dbody(t, carry):
        for k in range(CH // 16):
            idxv[pl.ds(k * 16, 16)] = dstv[t, pl.ds(k * 16, 16)] + s * NPAD
        pltpu.sync_copy(ones_v, deg_sp.at[idxv], add=True)
        return carry

    lax.fori_loop(0, NCHD, dbody, 0)
    plsc.subcore_barrier()
    for r in range(NSUB):
        pltpu.sync_copy(
            deg_sp.at[pl.ds(r * NPAD + wid * RPT, RPT)],
            sumv.at[pl.ds(r * RPT, RPT)],
        )
    for k in range(RPT // 16):
        acc = sumv[pl.ds(k * 16, 16)]
        for r in range(1, NSUB):
            acc = acc + sumv[pl.ds(r * RPT + k * 16, 16)]
        dv[pl.ds(k * 16, 16)] = acc
    pltpu.sync_copy(dv, deg_hbm.at[pl.ds(wid * RPT, RPT)])


@functools.partial(
    pl.kernel,
    out_type=jax.ShapeDtypeStruct((NCORES, NPAD, M), jnp.float32),
    mesh=_mesh,
    scratch_types=[
        pltpu.VMEM_SHARED((NPAD, M), jnp.float32),
        pltpu.VMEM((2, CH), jnp.int32),
        pltpu.VMEM((2, CH), jnp.int32),
        pltpu.VMEM((CH, M), jnp.float32),
        pltpu.VMEM((CH, M), jnp.float32),
        pltpu.SemaphoreType.DMA,
        pltpu.SemaphoreType.DMA,
    ],
)
def _hop(g_hbm, src_hbm, dst_hbm, out_hbm, acc_sp, sidx, didx, buf_a, buf_b, sem_a, sem_b):
    c = lax.axis_index("c")
    s = lax.axis_index("s")
    wid = c * NSUB + s
    for r in range(32):
        for k in range(M // 16):
            buf_a[r, pl.ds(k * 16, 16)] = jnp.zeros((16,), jnp.float32)
    for j in range(NPAD // NSUB // 32):
        base = s * (NPAD // NSUB) + j * 32
        pltpu.sync_copy(buf_a.at[pl.ds(0, 32)], acc_sp.at[pl.ds(base, 32)])
    plsc.subcore_barrier()

    bufs = (buf_a, buf_b)
    sems = (sem_a, sem_b)
    pltpu.sync_copy(src_hbm.at[wid, 0], sidx.at[0])
    pltpu.sync_copy(dst_hbm.at[wid, 0], didx.at[0])
    pltpu.async_copy(g_hbm.at[sidx.at[0]], buf_a, sem_a)

    def gbody(g, carry):
        for b in range(GRP):
            t = g * GRP + b
            cur_s, nxt_s = b % 2, (b + 1) % 2

            @pl.when(t + 1 < NCH)
            def _prefetch_idx():
                pltpu.sync_copy(src_hbm.at[wid, t + 1], sidx.at[nxt_s])
                pltpu.sync_copy(dst_hbm.at[wid, t + 1], didx.at[nxt_s])

            pltpu.make_async_copy(
                g_hbm.at[sidx.at[cur_s]], bufs[cur_s], sems[cur_s]
            ).wait()

            @pl.when(t + 1 < NCH)
            def _prefetch_rows():
                pltpu.async_copy(g_hbm.at[sidx.at[nxt_s]], bufs[nxt_s], sems[nxt_s])

            pltpu.sync_copy(bufs[cur_s], acc_sp.at[didx.at[cur_s]], add=True)
        return carry

    lax.fori_loop(0, NCH // GRP, gbody, 0)
    plsc.subcore_barrier()
    for j in range(NPAD // NSUB // 64):
        base = s * (NPAD // NSUB) + j * 64
        pltpu.sync_copy(acc_sp.at[pl.ds(base, 64)], buf_a.at[pl.ds(0, 64)])
        pltpu.sync_copy(buf_a.at[pl.ds(0, 64)], out_hbm.at[c, pl.ds(base, 64)])


BLK = 1280


def _scale_body(x_ref, deg_ref, g0_ref, dv_ref):
    dv = lax.rsqrt(deg_ref[...] + 1.0)
    dv_ref[...] = dv
    g0_ref[...] = dv * x_ref[...]


_scale = pl.pallas_call(
    _scale_body,
    grid=(NPAD // BLK,),
    in_specs=[
        pl.BlockSpec((BLK, M), lambda i: (i, 0)),
        pl.BlockSpec((BLK, 1), lambda i: (i, 0)),
    ],
    out_specs=[
        pl.BlockSpec((BLK, M), lambda i: (i, 0)),
        pl.BlockSpec((BLK, 1), lambda i: (i, 0)),
    ],
    out_shape=[
        jax.ShapeDtypeStruct((NPAD, M), jnp.float32),
        jax.ShapeDtypeStruct((NPAD, 1), jnp.float32),
    ],
)


def _combine_body(p_ref, g0_ref, dv_ref, o_ref):
    dv = dv_ref[...]
    o_ref[...] = dv * dv * (p_ref[0] + p_ref[1] + g0_ref[...])


_combine = pl.pallas_call(
    _combine_body,
    grid=(NPAD // BLK,),
    in_specs=[
        pl.BlockSpec((NCORES, BLK, M), lambda i: (0, i, 0)),
        pl.BlockSpec((BLK, M), lambda i: (i, 0)),
        pl.BlockSpec((BLK, 1), lambda i: (i, 0)),
    ],
    out_specs=pl.BlockSpec((BLK, M), lambda i: (i, 0)),
    out_shape=jax.ShapeDtypeStruct((NPAD, M), jnp.float32),
)


def _final_body(q_ref, g1_ref, dv_ref, w_ref, b_ref, o_ref):
    dv = dv_ref[...]
    h = dv * (q_ref[0] + q_ref[1] + g1_ref[...])
    o_ref[...] = (
        jnp.dot(h, w_ref[...], preferred_element_type=jnp.float32) + b_ref[...]
    )


_final = pl.pallas_call(
    _final_body,
    grid=(NPAD // BLK,),
    in_specs=[
        pl.BlockSpec((NCORES, BLK, M), lambda i: (0, i, 0)),
        pl.BlockSpec((BLK, M), lambda i: (i, 0)),
        pl.BlockSpec((BLK, 1), lambda i: (i, 0)),
        pl.BlockSpec((M, M), lambda i: (0, 0)),
        pl.BlockSpec((1, M), lambda i: (0, 0)),
    ],
    out_specs=pl.BlockSpec((BLK, M), lambda i: (i, 0)),
    out_shape=jax.ShapeDtypeStruct((NPAD, M), jnp.float32),
)


def kernel(x, edge_index, W, b):
    e = edge_index.shape[1]
    src = edge_index[0]
    dst = edge_index[1]
    pad = jnp.full((EPAD - e,), NPAD - 1, jnp.int32)
    srcp = jnp.concatenate([src, pad]).reshape(NW, NCH, CH)
    dstp = jnp.concatenate([dst, pad]).reshape(NW, NCH, CH)
    dstd = dstp.reshape(NSUB, NCHD, CH)
    xp = jnp.pad(x, ((0, NPAD - N), (0, 0)))

    deg = _deg(dstd)
    g0, dcol = _scale(xp, deg.reshape(NPAD, 1))
    p = _hop(g0, srcp, dstp)
    g1 = _combine(p, g0, dcol)
    q = _hop(g1, srcp, dstp)
    out = _final(q, g1, dcol, W, b.reshape(1, M))
    return out[:N]


# speedup vs baseline: 9.2233x; 1.0003x over previous
import functools

import jax
import jax.numpy as jnp
from jax import lax
from jax.experimental import pallas as pl
from jax.experimental.pallas import tpu as pltpu
from jax.experimental.pallas import tpu_sc as plsc

N = 10000
M = 128
NPAD = 10240
NCORES = 2
NSUB = 16
NW = NCORES * NSUB
RPT = NPAD // NW
CH = 128
EPT = 10240
NCH = EPT // CH
EPAD = NW * EPT
NCHD = 2 * NCH
GRP = 8

_mesh = plsc.VectorSubcoreMesh(
    core_axis_name="c", subcore_axis_name="s", num_cores=NCORES, num_subcores=NSUB
)


@functools.partial(
    pl.kernel,
    out_type=jax.ShapeDtypeStruct((NPAD,), jnp.float32),
    mesh=_mesh,
    scratch_types=[
        pltpu.VMEM_SHARED((NSUB * NPAD,), jnp.float32),
        pltpu.VMEM((NCHD, CH), jnp.int32),
        pltpu.VMEM((CH,), jnp.int32),
        pltpu.VMEM((CH,), jnp.float32),
        pltpu.VMEM((RPT,), jnp.float32),
        pltpu.VMEM((NSUB * RPT,), jnp.float32),
    ],
)
def _deg(dstd_hbm, deg_hbm, deg_sp, dstv, idxv, ones_v, dv, sumv):
    c = lax.axis_index("c")
    s = lax.axis_index("s")
    wid = c * NSUB + s
    for k in range(RPT // 16):
        dv[pl.ds(k * 16, 16)] = jnp.zeros((16,), jnp.float32)
    for j in range(NPAD // RPT):
        pltpu.sync_copy(dv, deg_sp.at[pl.ds(s * NPAD + j * RPT, RPT)])
    for k in range(CH // 16):
        ones_v[pl.ds(k * 16, 16)] = jnp.ones((16,), jnp.float32)
    pltpu.sync_copy(dstd_hbm.at[s], dstv)

    def dbody(t, carry):
        for k in range(CH // 16):
            idxv[pl.ds(k * 16, 16)] = dstv[t, pl.ds(k * 16, 16)] + s * NPAD
        pltpu.sync_copy(ones_v, deg_sp.at[idxv], add=True)
        return carry

    lax.fori_loop(0, NCHD, dbody, 0)
    plsc.subcore_barrier()
    for r in range(NSUB):
        pltpu.sync_copy(
            deg_sp.at[pl.ds(r * NPAD + wid * RPT, RPT)],
            sumv.at[pl.ds(r * RPT, RPT)],
        )
    for k in range(RPT // 16):
        acc = sumv[pl.ds(k * 16, 16)]
        for r in range(1, NSUB):
            acc = acc + sumv[pl.ds(r * RPT + k * 16, 16)]
        dv[pl.ds(k * 16, 16)] = acc
    pltpu.sync_copy(dv, deg_hbm.at[pl.ds(wid * RPT, RPT)])


@functools.partial(
    pl.kernel,
    out_type=jax.ShapeDtypeStruct((NCORES, NPAD, M), jnp.float32),
    mesh=_mesh,
    scratch_types=[
        pltpu.VMEM_SHARED((NPAD, M), jnp.float32),
        pltpu.VMEM((2, CH), jnp.int32),
        pltpu.VMEM((2, CH), jnp.int32),
        pltpu.VMEM((CH, M), jnp.float32),
        pltpu.VMEM((CH, M), jnp.float32),
        pltpu.SemaphoreType.DMA,
        pltpu.SemaphoreType.DMA,
    ],
)
def _hop(g_hbm, src_hbm, dst_hbm, out_hbm, acc_sp, sidx, didx, buf_a, buf_b, sem_a, sem_b):
    c = lax.axis_index("c")
    s = lax.axis_index("s")
    wid = c * NSUB + s
    for r in range(32):
        for k in range(M // 16):
            buf_a[r, pl.ds(k * 16, 16)] = jnp.zeros((16,), jnp.float32)
    for j in range(NPAD // NSUB // 32):
        base = s * (NPAD // NSUB) + j * 32
        pltpu.sync_copy(buf_a.at[pl.ds(0, 32)], acc_sp.at[pl.ds(base, 32)])
    plsc.subcore_barrier()

    bufs = (buf_a, buf_b)
    sems = (sem_a, sem_b)
    pltpu.sync_copy(src_hbm.at[wid, 0], sidx.at[0])
    pltpu.sync_copy(dst_hbm.at[wid, 0], didx.at[0])
    pltpu.async_copy(g_hbm.at[sidx.at[0]], buf_a, sem_a)

    def gbody(g, carry):
        for b in range(GRP):
            t = g * GRP + b
            cur_s, nxt_s = b % 2, (b + 1) % 2

            @pl.when(t + 1 < NCH)
            def _prefetch_idx():
                pltpu.sync_copy(src_hbm.at[wid, t + 1], sidx.at[nxt_s])
                pltpu.sync_copy(dst_hbm.at[wid, t + 1], didx.at[nxt_s])

            pltpu.make_async_copy(
                g_hbm.at[sidx.at[cur_s]], bufs[cur_s], sems[cur_s]
            ).wait()

            @pl.when(t + 1 < NCH)
            def _prefetch_rows():
                pltpu.async_copy(g_hbm.at[sidx.at[nxt_s]], bufs[nxt_s], sems[nxt_s])

            pltpu.sync_copy(bufs[cur_s], acc_sp.at[didx.at[cur_s]], add=True)
        return carry

    lax.fori_loop(0, NCH // GRP, gbody, 0)
    plsc.subcore_barrier()
    for j in range(NPAD // NSUB // 64):
        base = s * (NPAD // NSUB) + j * 64
        pltpu.sync_copy(acc_sp.at[pl.ds(base, 64)], buf_a.at[pl.ds(0, 64)])
        pltpu.sync_copy(buf_a.at[pl.ds(0, 64)], out_hbm.at[c, pl.ds(base, 64)])


BLK = 1280


def _scale_body(x_ref, deg_ref, g0_ref, dv_ref):
    dv = lax.rsqrt(deg_ref[...] + 1.0)
    dv_ref[...] = dv
    g0_ref[...] = dv * x_ref[...]


_scale = pl.pallas_call(
    _scale_body,
    grid=(NPAD // BLK,),
    in_specs=[
        pl.BlockSpec((BLK, M), lambda i: (i, 0)),
        pl.BlockSpec((BLK, 1), lambda i: (i, 0)),
    ],
    out_specs=[
        pl.BlockSpec((BLK, M), lambda i: (i, 0)),
        pl.BlockSpec((BLK, 1), lambda i: (i, 0)),
    ],
    out_shape=[
        jax.ShapeDtypeStruct((NPAD, M), jnp.float32),
        jax.ShapeDtypeStruct((NPAD, 1), jnp.float32),
    ],
)


def _combine_body(p_ref, g0_ref, dv_ref, o_ref):
    dv = dv_ref[...]
    o_ref[...] = dv * dv * (p_ref[0] + p_ref[1] + g0_ref[...])


_combine = pl.pallas_call(
    _combine_body,
    grid=(NPAD // BLK,),
    in_specs=[
        pl.BlockSpec((NCORES, BLK, M), lambda i: (0, i, 0)),
        pl.BlockSpec((BLK, M), lambda i: (i, 0)),
        pl.BlockSpec((BLK, 1), lambda i: (i, 0)),
    ],
    out_specs=pl.BlockSpec((BLK, M), lambda i: (i, 0)),
    out_shape=jax.ShapeDtypeStruct((NPAD, M), jnp.float32),
)


def _final_body(q_ref, g1_ref, dv_ref, w_ref, b_ref, o_ref):
    dv = dv_ref[...]
    h = dv * (q_ref[0] + q_ref[1] + g1_ref[...])
    o_ref[...] = (
        jnp.dot(h, w_ref[...], preferred_element_type=jnp.float32) + b_ref[...]
    )


_final = pl.pallas_call(
    _final_body,
    grid=(NPAD // BLK,),
    in_specs=[
        pl.BlockSpec((NCORES, BLK, M), lambda i: (0, i, 0)),
        pl.BlockSpec((BLK, M), lambda i: (i, 0)),
        pl.BlockSpec((BLK, 1), lambda i: (i, 0)),
        pl.BlockSpec((M, M), lambda i: (0, 0)),
        pl.BlockSpec((1, M), lambda i: (0, 0)),
    ],
    out_specs=pl.BlockSpec((BLK, M), lambda i: (i, 0)),
    out_shape=jax.ShapeDtypeStruct((NPAD, M), jnp.float32),
)


def kernel(x, edge_index, W, b):
    e = edge_index.shape[1]
    src = edge_index[0]
    dst = edge_index[1]
    npad_rows = NPAD - N
    pad_src = jnp.full((EPAD - e,), NPAD - 1, jnp.int32)
    pad_dst = N + (jnp.arange(EPAD - e, dtype=jnp.int32) % npad_rows)
    srcp = jnp.concatenate([src, pad_src]).reshape(NW, NCH, CH)
    dstp = jnp.concatenate([dst, pad_dst]).reshape(NW, NCH, CH)
    dstd = dstp.reshape(NSUB, NCHD, CH)
    xp = jnp.pad(x, ((0, NPAD - N), (0, 0)))

    deg = _deg(dstd)
    g0, dcol = _scale(xp, deg.reshape(NPAD, 1))
    p = _hop(g0, srcp, dstp)
    g1 = _combine(p, g0, dcol)
    q = _hop(g1, srcp, dstp)
    out = _final(q, g1, dcol, W, b.reshape(1, M))
    return out[:N]
